# Initial kernel scaffold; baseline (speedup 1.0000x reference)
import jax
import jax.numpy as jnp
from jax.experimental import pallas as pl


def kernel(feats, edge_index, W, b):
    raise NotImplementedError("write your pallas kernel here")


# speedup vs baseline: 4.3155x; 4.3155x over previous
import functools

import jax
import jax.numpy as jnp
from jax import lax
from jax.experimental import pallas as pl
from jax.experimental.pallas import tpu as pltpu
from jax.experimental.pallas import tpu_sc as plsc

N = 10000
E = 320000
F = 128
DH = F // 2
NC = 2
NS = 16
CH = 128
STRIPE = 640
N_PAD = NS * STRIPE
TRASH = N
PCH = 160
DCH = PCH // 2
E_PAD = NS * PCH * CH

_mesh = plsc.VectorSubcoreMesh(core_axis_name="c", subcore_axis_name="s")


@functools.partial(
    pl.kernel,
    out_type=jax.ShapeDtypeStruct((NC, N_PAD, 16), jnp.float32),
    mesh=_mesh,
    scratch_types=[
        pltpu.VMEM((DCH, CH), jnp.int32),
        pltpu.VMEM((CH, 16), jnp.float32),
        pltpu.VMEM_SHARED((N_PAD, 16), jnp.float32),
    ],
    compiler_params=pltpu.CompilerParams(use_tc_tiling_on_sc=False),
)
def _deg_kernel(dst_hbm, zeros_hbm, ones_hbm, out_hbm, idx_v, ones_v, acc_sh):
    cid = lax.axis_index("c")
    sid = lax.axis_index("s")
    base = sid * STRIPE
    pltpu.sync_copy(zeros_hbm.at[pl.ds(base, STRIPE)],
                    acc_sh.at[pl.ds(base, STRIPE)])
    pltpu.sync_copy(ones_hbm, ones_v)
    pltpu.sync_copy(dst_hbm.at[pl.ds((sid * NC + cid) * DCH, DCH)], idx_v)
    plsc.subcore_barrier()

    def body(j, carry):
        pltpu.sync_copy(ones_v, acc_sh.at[idx_v.at[j]], add=True)
        return carry

    lax.fori_loop(0, DCH, body, 0)
    plsc.subcore_barrier()
    pltpu.sync_copy(acc_sh.at[pl.ds(base, STRIPE)],
                    out_hbm.at[cid, pl.ds(base, STRIPE)])


@functools.partial(
    pl.kernel,
    out_type=jax.ShapeDtypeStruct((NC * N_PAD, DH), jnp.float32),
    mesh=_mesh,
    scratch_types=[
        pltpu.VMEM((PCH, CH), jnp.int32),
        pltpu.VMEM((PCH, CH), jnp.int32),
        pltpu.VMEM((CH, DH), jnp.float32),
        pltpu.VMEM((CH, DH), jnp.float32),
        pltpu.VMEM_SHARED((N_PAD, DH), jnp.float32),
        pltpu.SemaphoreType.DMA,
        pltpu.SemaphoreType.DMA,
    ],
    compiler_params=pltpu.CompilerParams(use_tc_tiling_on_sc=False),
)
def _prop_kernel(hn_hbm, src_hbm, dst_hbm, zeros_hbm, out_hbm,
                 src_v, dst_v, rows0, rows1, acc_sh, sem0, sem1):
    cid = lax.axis_index("c")
    sid = lax.axis_index("s")
    base = sid * STRIPE
    pltpu.sync_copy(zeros_hbm.at[pl.ds(base, STRIPE)],
                    acc_sh.at[pl.ds(base, STRIPE)])
    pltpu.sync_copy(src_hbm.at[cid, pl.ds(sid * PCH, PCH)], src_v)
    pltpu.sync_copy(dst_hbm.at[pl.ds(sid * PCH, PCH)], dst_v)
    plsc.subcore_barrier()

    pltpu.async_copy(hn_hbm.at[src_v.at[0]], rows0, sem0)
    pltpu.async_copy(hn_hbm.at[src_v.at[1]], rows1, sem1)

    def body(jj, carry):
        c0 = 2 * jj
        pltpu.make_async_copy(hn_hbm.at[pl.ds(0, CH)], rows0, sem0).wait()
        pltpu.sync_copy(rows0, acc_sh.at[dst_v.at[c0]], add=True)

        @pl.when(c0 + 2 < PCH)
        def _():
            pltpu.async_copy(hn_hbm.at[src_v.at[c0 + 2]], rows0, sem0)

        pltpu.make_async_copy(hn_hbm.at[pl.ds(0, CH)], rows1, sem1).wait()
        pltpu.sync_copy(rows1, acc_sh.at[dst_v.at[c0 + 1]], add=True)

        @pl.when(c0 + 3 < PCH)
        def _():
            pltpu.async_copy(hn_hbm.at[src_v.at[c0 + 3]], rows1, sem1)

        return carry

    lax.fori_loop(0, PCH // 2, body, 0)
    plsc.subcore_barrier()
    pltpu.sync_copy(acc_sh.at[pl.ds(base, STRIPE)],
                    out_hbm.at[pl.ds(cid * N_PAD + base, STRIPE)])


def _prep_body(deg2_ref, feats_ref, norm_ref, hn0_ref):
    deg = deg2_ref[0, :, 0:1] + deg2_ref[1, :, 0:1]
    norm = lax.rsqrt(jnp.maximum(deg, 1.0))
    normb = lax.broadcast_in_dim(norm, (N_PAD, DH), (0, 1))
    norm_ref[...] = normb
    nb = normb[:N, :]
    hn0_ref[pl.ds(0, N), :] = feats_ref[:, :DH] * nb
    hn0_ref[pl.ds(N_PAD, N), :] = feats_ref[:, DH:] * nb


_prep_call = pl.pallas_call(
    _prep_body,
    out_shape=(
        jax.ShapeDtypeStruct((N_PAD, DH), jnp.float32),
        jax.ShapeDtypeStruct((NC * N_PAD, DH), jnp.float32),
    ),
)


def _scale_body(a_ref, norm_ref, hn_ref):
    n2 = norm_ref[...] * norm_ref[...]
    hn_ref[pl.ds(0, N_PAD), :] = a_ref[pl.ds(0, N_PAD), :] * n2
    hn_ref[pl.ds(N_PAD, N_PAD), :] = a_ref[pl.ds(N_PAD, N_PAD), :] * n2


_scale_call = pl.pallas_call(
    _scale_body,
    out_shape=jax.ShapeDtypeStruct((NC * N_PAD, DH), jnp.float32),
)


MM_R = 1280


def _mm_body(feats_ref, a1p0, a1p1, a2p0, a2p1, norm_ref, w_ref, b_ref,
             out_ref):
    nb = norm_ref[...]
    dot = functools.partial(jnp.dot, preferred_element_type=jnp.float32,
                            precision=lax.Precision.HIGHEST)
    acc = dot(feats_ref[...], w_ref[pl.ds(0, F), :])
    acc += dot(a1p0[0] * nb, w_ref[pl.ds(F, DH), :])
    acc += dot(a1p1[0] * nb, w_ref[pl.ds(F + DH, DH), :])
    acc += dot(a2p0[0] * nb, w_ref[pl.ds(2 * F, DH), :])
    acc += dot(a2p1[0] * nb, w_ref[pl.ds(2 * F + DH, DH), :])
    out_ref[...] = acc + b_ref[...]


_mm_call = pl.pallas_call(
    _mm_body,
    grid=(N_PAD // MM_R,),
    in_specs=[
        pl.BlockSpec((MM_R, F), lambda i: (i, 0)),
        pl.BlockSpec((1, MM_R, DH), lambda i: (0, i, 0)),
        pl.BlockSpec((1, MM_R, DH), lambda i: (1, i, 0)),
        pl.BlockSpec((1, MM_R, DH), lambda i: (0, i, 0)),
        pl.BlockSpec((1, MM_R, DH), lambda i: (1, i, 0)),
        pl.BlockSpec((MM_R, DH), lambda i: (i, 0)),
        pl.BlockSpec((3 * F, F), lambda i: (0, 0)),
        pl.BlockSpec((1, F), lambda i: (0, 0)),
    ],
    out_specs=pl.BlockSpec((MM_R, F), lambda i: (i, 0)),
    out_shape=jax.ShapeDtypeStruct((N_PAD, F), jnp.float32),
)


def kernel(feats, edge_index, W, b):
    src = edge_index[0].astype(jnp.int32)
    dst = edge_index[1].astype(jnp.int32)

    src_pad = jnp.zeros((E_PAD,), jnp.int32).at[:E].set(src)
    dst_c = jnp.full((E_PAD,), TRASH, jnp.int32).at[:E].set(dst).reshape(-1, CH)
    src_pc = (src_pad[None, :]
              + (jnp.arange(NC, dtype=jnp.int32) * N_PAD)[:, None]
              ).reshape(NC, NS * PCH, CH)

    zeros = jnp.zeros((N_PAD, DH), jnp.float32)
    zeros16 = jnp.zeros((N_PAD, 16), jnp.float32)
    ones16 = jnp.ones((CH, 16), jnp.float32)

    deg2 = _deg_kernel(dst_c, zeros16, ones16)
    normb, hn0 = _prep_call(deg2, feats)
    a1 = _prop_kernel(hn0, src_pc, dst_c, zeros)
    hn1 = _scale_call(a1, normb)
    a2 = _prop_kernel(hn1, src_pc, dst_c, zeros)

    feats_pad = jnp.zeros((N_PAD, F), jnp.float32).at[:N].set(feats)
    a1r = a1.reshape(NC, N_PAD, DH)
    a2r = a2.reshape(NC, N_PAD, DH)
    out = _mm_call(feats_pad, a1r, a1r, a2r, a2r, normb, W, b[None, :])
    return out[:N]


# speedup vs baseline: 4.5389x; 1.0518x over previous
import functools

import jax
import jax.numpy as jnp
from jax import lax
from jax.experimental import pallas as pl
from jax.experimental.pallas import tpu as pltpu
from jax.experimental.pallas import tpu_sc as plsc

N = 10000
E = 320000
F = 128
DH = F // 2
NC = 2
NS = 16
CH = 128
STRIPE = 640
N_PAD = NS * STRIPE
TRASH = N
PCH = 160
DCH = PCH // 2
E_PAD = NS * PCH * CH

_mesh = plsc.VectorSubcoreMesh(core_axis_name="c", subcore_axis_name="s")
_sc_params = pltpu.CompilerParams(use_tc_tiling_on_sc=False)


@functools.partial(
    pl.kernel,
    out_type=jax.ShapeDtypeStruct((NC, N_PAD, 16), jnp.float32),
    mesh=_mesh,
    scratch_types=[
        pltpu.VMEM((DCH, CH), jnp.int32),
        pltpu.VMEM((CH, 16), jnp.float32),
        pltpu.VMEM_SHARED((N_PAD, 16), jnp.float32),
    ],
    compiler_params=_sc_params,
)
def _deg_kernel(dst_hbm, zeros_hbm, ones_hbm, out_hbm, idx_v, ones_v, acc_sh):
    cid = lax.axis_index("c")
    sid = lax.axis_index("s")
    base = sid * STRIPE
    pltpu.sync_copy(zeros_hbm.at[pl.ds(base, STRIPE)],
                    acc_sh.at[pl.ds(base, STRIPE)])
    pltpu.sync_copy(ones_hbm, ones_v)
    pltpu.sync_copy(dst_hbm.at[pl.ds((sid * NC + cid) * DCH, DCH)], idx_v)
    plsc.subcore_barrier()

    def body(j, carry):
        pltpu.sync_copy(ones_v, acc_sh.at[idx_v.at[j]], add=True)
        return carry

    lax.fori_loop(0, DCH, body, 0)
    plsc.subcore_barrier()
    pltpu.sync_copy(acc_sh.at[pl.ds(base, STRIPE)],
                    out_hbm.at[cid, pl.ds(base, STRIPE)])


@functools.partial(
    pl.kernel,
    out_type=jax.ShapeDtypeStruct((NC, N_PAD, DH), jnp.float32),
    mesh=_mesh,
    scratch_types=[
        pltpu.VMEM((PCH, CH), jnp.int32),
        pltpu.VMEM((PCH, CH), jnp.int32),
        [pltpu.VMEM((CH, DH), jnp.float32)] * 4,
        pltpu.VMEM_SHARED((N_PAD, DH), jnp.float32),
        [pltpu.SemaphoreType.DMA] * 4,
        [pltpu.SemaphoreType.DMA] * 4,
    ],
    compiler_params=_sc_params,
)
def _prop_kernel(hn_hbm, src_hbm, dst_hbm, zeros_hbm, out_hbm,
                 src_v, dst_v, rows, acc_sh, gsems, ssems):
    cid = lax.axis_index("c")
    sid = lax.axis_index("s")
    base = sid * STRIPE
    table = hn_hbm.at[cid]
    pltpu.sync_copy(zeros_hbm.at[pl.ds(base, STRIPE)],
                    acc_sh.at[pl.ds(base, STRIPE)])
    pltpu.sync_copy(src_hbm.at[pl.ds(sid * PCH, PCH)], src_v)
    pltpu.sync_copy(dst_hbm.at[pl.ds(sid * PCH, PCH)], dst_v)
    plsc.subcore_barrier()

    pltpu.async_copy(table.at[src_v.at[0]], rows[0], gsems[0])
    pltpu.async_copy(table.at[src_v.at[1]], rows[1], gsems[1])

    def drain(sem, buf):
        pltpu.make_async_copy(table.at[pl.ds(0, CH)], buf, sem).wait()

    def step(c, k):
        drain(gsems[k], rows[k])
        pltpu.async_copy(rows[k], acc_sh.at[dst_v.at[c]], ssems[k],
                         add=True)
        k2 = (k + 2) % 4

        @pl.when(c >= 2)
        def _():
            drain(ssems[k2], rows[k2])

        @pl.when(c + 2 < PCH)
        def _():
            pltpu.async_copy(table.at[src_v.at[c + 2]], rows[k2], gsems[k2])

    def body(jj, carry):
        c0 = 4 * jj
        for k in range(4):
            step(c0 + k, k)
        return carry

    lax.fori_loop(0, PCH // 4, body, 0)
    drain(ssems[(PCH - 2) % 4], rows[(PCH - 2) % 4])
    drain(ssems[(PCH - 1) % 4], rows[(PCH - 1) % 4])
    plsc.subcore_barrier()
    pltpu.sync_copy(acc_sh.at[pl.ds(base, STRIPE)],
                    out_hbm.at[cid, pl.ds(base, STRIPE)])


def _prep_body(deg2_ref, feats_ref, norm_ref, hn0_ref):
    deg = deg2_ref[0, :, 0:1] + deg2_ref[1, :, 0:1]
    norm = lax.rsqrt(jnp.maximum(deg, 1.0))
    normb = lax.broadcast_in_dim(norm, (N_PAD, DH), (0, 1))
    norm_ref[...] = normb
    nb = normb[:N, :]
    hn0_ref[0, pl.ds(0, N), :] = feats_ref[:, :DH] * nb
    hn0_ref[1, pl.ds(0, N), :] = feats_ref[:, DH:] * nb


_prep_call = pl.pallas_call(
    _prep_body,
    out_shape=(
        jax.ShapeDtypeStruct((N_PAD, DH), jnp.float32),
        jax.ShapeDtypeStruct((NC, N_PAD, DH), jnp.float32),
    ),
)


def _scale_body(a_ref, norm_ref, hn_ref):
    n2 = norm_ref[...] * norm_ref[...]
    hn_ref[0] = a_ref[0] * n2
    hn_ref[1] = a_ref[1] * n2


_scale_call = pl.pallas_call(
    _scale_body,
    out_shape=jax.ShapeDtypeStruct((NC, N_PAD, DH), jnp.float32),
)


MM_R = 2000


def _mm_body(feats_ref, a1p0, a1p1, a2p0, a2p1, norm_ref, w_ref, b_ref,
             out_ref):
    nb = norm_ref[...]
    dot = functools.partial(jnp.dot, preferred_element_type=jnp.float32,
                            precision=lax.Precision.HIGHEST)
    acc = dot(feats_ref[...], w_ref[pl.ds(0, F), :])
    acc += dot(a1p0[0] * nb, w_ref[pl.ds(F, DH), :])
    acc += dot(a1p1[0] * nb, w_ref[pl.ds(F + DH, DH), :])
    acc += dot(a2p0[0] * nb, w_ref[pl.ds(2 * F, DH), :])
    acc += dot(a2p1[0] * nb, w_ref[pl.ds(2 * F + DH, DH), :])
    out_ref[...] = acc + b_ref[...]


_mm_call = pl.pallas_call(
    _mm_body,
    grid=(N // MM_R,),
    in_specs=[
        pl.BlockSpec((MM_R, F), lambda i: (i, 0)),
        pl.BlockSpec((1, MM_R, DH), lambda i: (0, i, 0)),
        pl.BlockSpec((1, MM_R, DH), lambda i: (1, i, 0)),
        pl.BlockSpec((1, MM_R, DH), lambda i: (0, i, 0)),
        pl.BlockSpec((1, MM_R, DH), lambda i: (1, i, 0)),
        pl.BlockSpec((MM_R, DH), lambda i: (i, 0)),
        pl.BlockSpec((3 * F, F), lambda i: (0, 0)),
        pl.BlockSpec((1, F), lambda i: (0, 0)),
    ],
    out_specs=pl.BlockSpec((MM_R, F), lambda i: (i, 0)),
    out_shape=jax.ShapeDtypeStruct((N, F), jnp.float32),
)


def kernel(feats, edge_index, W, b):
    src = edge_index[0].astype(jnp.int32)
    dst = edge_index[1].astype(jnp.int32)

    src_c = jnp.zeros((E_PAD,), jnp.int32).at[:E].set(src).reshape(-1, CH)
    dst_c = jnp.full((E_PAD,), TRASH, jnp.int32).at[:E].set(dst).reshape(-1, CH)

    zeros = jnp.zeros((N_PAD, DH), jnp.float32)
    zeros16 = jnp.zeros((N_PAD, 16), jnp.float32)
    ones16 = jnp.ones((CH, 16), jnp.float32)

    deg2 = _deg_kernel(dst_c, zeros16, ones16)
    normb, hn0 = _prep_call(deg2, feats)
    a1 = _prop_kernel(hn0, src_c, dst_c, zeros)
    hn1 = _scale_call(a1, normb)
    a2 = _prop_kernel(hn1, src_c, dst_c, zeros)
    return _mm_call(feats, a1, a1, a2, a2, normb, W, b[None, :])


# speedup vs baseline: 4.9494x; 1.0904x over previous
import functools

import jax
import jax.numpy as jnp
from jax import lax
from jax.experimental import pallas as pl
from jax.experimental.pallas import tpu as pltpu
from jax.experimental.pallas import tpu_sc as plsc

N = 10000
E = 320000
F = 128
DH = F // 2
NC = 2
NS = 16
CH = 128
STRIPE = 632
N_PAD = NS * STRIPE
TRASH = N
CHP = 256
PCH = 80
DCH = 80
E_PAD = NS * PCH * CHP

_mesh = plsc.VectorSubcoreMesh(core_axis_name="c", subcore_axis_name="s")
_sc_params = pltpu.CompilerParams(use_tc_tiling_on_sc=False)


@functools.partial(
    pl.kernel,
    out_type=jax.ShapeDtypeStruct((NC, N_PAD, 16), jnp.float32),
    mesh=_mesh,
    scratch_types=[
        pltpu.VMEM((DCH, CH), jnp.int32),
        pltpu.VMEM((CH, 16), jnp.float32),
        pltpu.VMEM_SHARED((N_PAD, 16), jnp.float32),
    ],
    compiler_params=_sc_params,
)
def _deg_kernel(dst_hbm, zeros_hbm, ones_hbm, out_hbm, idx_v, ones_v, acc_sh):
    cid = lax.axis_index("c")
    sid = lax.axis_index("s")
    base = sid * STRIPE
    pltpu.sync_copy(zeros_hbm.at[pl.ds(base, STRIPE)],
                    acc_sh.at[pl.ds(base, STRIPE)])
    pltpu.sync_copy(ones_hbm, ones_v)
    pltpu.sync_copy(dst_hbm.at[pl.ds((sid * NC + cid) * DCH, DCH)], idx_v)
    plsc.subcore_barrier()

    def body(j, carry):
        pltpu.sync_copy(ones_v, acc_sh.at[idx_v.at[j]], add=True)
        return carry

    lax.fori_loop(0, DCH, body, 0)
    plsc.subcore_barrier()
    pltpu.sync_copy(acc_sh.at[pl.ds(base, STRIPE)],
                    out_hbm.at[cid, pl.ds(base, STRIPE)])


@functools.partial(
    pl.kernel,
    out_type=jax.ShapeDtypeStruct((NC, N_PAD, DH), jnp.float32),
    mesh=_mesh,
    scratch_types=[
        pltpu.VMEM((PCH, CHP), jnp.int32),
        pltpu.VMEM((PCH, CHP), jnp.int32),
        [pltpu.VMEM((CHP, DH), jnp.float32)] * 3,
        pltpu.VMEM_SHARED((N_PAD, DH), jnp.float32),
        [pltpu.SemaphoreType.DMA] * 3,
        [pltpu.SemaphoreType.DMA] * 3,
    ],
    compiler_params=_sc_params,
)
def _prop_kernel(hn_hbm, src_hbm, dst_hbm, zeros_hbm, out_hbm,
                 src_v, dst_v, rows, acc_sh, gsems, ssems):
    cid = lax.axis_index("c")
    sid = lax.axis_index("s")
    base = sid * STRIPE
    table = hn_hbm.at[cid]
    pltpu.sync_copy(zeros_hbm.at[pl.ds(base, STRIPE)],
                    acc_sh.at[pl.ds(base, STRIPE)])
    pltpu.sync_copy(src_hbm.at[pl.ds(sid * PCH, PCH)], src_v)
    pltpu.sync_copy(dst_hbm.at[pl.ds(sid * PCH, PCH)], dst_v)
    plsc.subcore_barrier()

    pltpu.async_copy(table.at[src_v.at[0]], rows[0], gsems[0])

    def drain(sem, buf):
        pltpu.make_async_copy(table.at[pl.ds(0, CHP)], buf, sem).wait()

    def step(c, k):
        drain(gsems[k], rows[k])
        pltpu.async_copy(rows[k], acc_sh.at[dst_v.at[c]], ssems[k],
                         add=True)
        k1 = (k + 1) % 3

        @pl.when(c >= 2)
        def _():
            drain(ssems[k1], rows[k1])

        @pl.when(c + 1 < PCH)
        def _():
            pltpu.async_copy(table.at[src_v.at[c + 1]], rows[k1], gsems[k1])

    def body(jj, carry):
        c0 = 3 * jj
        for k in range(3):
            step(c0 + k, k)
        return carry

    lax.fori_loop(0, PCH // 3, body, 0)
    for c in range(3 * (PCH // 3), PCH):
        step(jnp.int32(c), c % 3)
    drain(ssems[(PCH - 2) % 3], rows[(PCH - 2) % 3])
    drain(ssems[(PCH - 1) % 3], rows[(PCH - 1) % 3])
    plsc.subcore_barrier()
    pltpu.sync_copy(acc_sh.at[pl.ds(base, STRIPE)],
                    out_hbm.at[cid, pl.ds(base, STRIPE)])


def _prep_body(deg2_ref, feats_ref, norm_ref, hn0_ref):
    deg = deg2_ref[0, :, 0:1] + deg2_ref[1, :, 0:1]
    norm = lax.rsqrt(jnp.maximum(deg, 1.0))
    normb = lax.broadcast_in_dim(norm, (N_PAD, DH), (0, 1))
    norm_ref[...] = normb
    nb = normb[:N, :]
    hn0_ref[0, pl.ds(0, N), :] = feats_ref[:, :DH] * nb
    hn0_ref[1, pl.ds(0, N), :] = feats_ref[:, DH:] * nb


_prep_call = pl.pallas_call(
    _prep_body,
    out_shape=(
        jax.ShapeDtypeStruct((N_PAD, DH), jnp.float32),
        jax.ShapeDtypeStruct((NC, N_PAD, DH), jnp.float32),
    ),
)


def _scale_body(a_ref, norm_ref, hn_ref):
    n2 = norm_ref[...] * norm_ref[...]
    hn_ref[0] = a_ref[0] * n2
    hn_ref[1] = a_ref[1] * n2


_scale_call = pl.pallas_call(
    _scale_body,
    out_shape=jax.ShapeDtypeStruct((NC, N_PAD, DH), jnp.float32),
)


MM_R = 2000


def _mm_body(feats_ref, a1p0, a1p1, a2p0, a2p1, norm_ref, w_ref, b_ref,
             out_ref):
    nb = norm_ref[...]
    dot = functools.partial(jnp.dot, preferred_element_type=jnp.float32,
                            precision=lax.Precision.HIGHEST)
    acc = dot(feats_ref[...], w_ref[pl.ds(0, F), :])
    acc += dot(a1p0[0] * nb, w_ref[pl.ds(F, DH), :])
    acc += dot(a1p1[0] * nb, w_ref[pl.ds(F + DH, DH), :])
    acc += dot(a2p0[0] * nb, w_ref[pl.ds(2 * F, DH), :])
    acc += dot(a2p1[0] * nb, w_ref[pl.ds(2 * F + DH, DH), :])
    out_ref[...] = acc + b_ref[...]


_mm_call = pl.pallas_call(
    _mm_body,
    grid=(N // MM_R,),
    in_specs=[
        pl.BlockSpec((MM_R, F), lambda i: (i, 0)),
        pl.BlockSpec((1, MM_R, DH), lambda i: (0, i, 0)),
        pl.BlockSpec((1, MM_R, DH), lambda i: (1, i, 0)),
        pl.BlockSpec((1, MM_R, DH), lambda i: (0, i, 0)),
        pl.BlockSpec((1, MM_R, DH), lambda i: (1, i, 0)),
        pl.BlockSpec((MM_R, DH), lambda i: (i, 0)),
        pl.BlockSpec((3 * F, F), lambda i: (0, 0)),
        pl.BlockSpec((1, F), lambda i: (0, 0)),
    ],
    out_specs=pl.BlockSpec((MM_R, F), lambda i: (i, 0)),
    out_shape=jax.ShapeDtypeStruct((N, F), jnp.float32),
)


def kernel(feats, edge_index, W, b):
    src = edge_index[0].astype(jnp.int32)
    dst = edge_index[1].astype(jnp.int32)

    src_flat = jnp.zeros((E_PAD,), jnp.int32).at[:E].set(src)
    dst_flat = jnp.full((E_PAD,), TRASH, jnp.int32).at[:E].set(dst)
    src_c = src_flat.reshape(-1, CHP)
    dst_c = dst_flat.reshape(-1, CHP)
    dst_c128 = dst_flat.reshape(-1, CH)

    zeros = jnp.zeros((N_PAD, DH), jnp.float32)
    zeros16 = jnp.zeros((N_PAD, 16), jnp.float32)
    ones16 = jnp.ones((CH, 16), jnp.float32)

    deg2 = _deg_kernel(dst_c128, zeros16, ones16)
    normb, hn0 = _prep_call(deg2, feats)
    a1 = _prop_kernel(hn0, src_c, dst_c, zeros)
    hn1 = _scale_call(a1, normb)
    a2 = _prop_kernel(hn1, src_c, dst_c, zeros)
    return _mm_call(feats, a1, a1, a2, a2, normb, W, b[None, :])
